# baseline (speedup 1.0000x reference)
import functools

import jax
import jax.numpy as jnp
from jax import lax
from jax.experimental import pallas as pl
from jax.experimental.pallas import tpu as pltpu
from jax.experimental.pallas import tpu_sc as plsc

NW = 32


@functools.lru_cache(maxsize=None)
def _make_prep(C, H, W, HB, CQ):
    S = H * W

    def body(q_ref, gx_ref, gy_ref, tq_ref, tg_ref):
        for h in range(HB):
            tq_ref[h * W:(h + 1) * W, :C] = q_ref[:, h, :].T
            tq_ref[h * W:(h + 1) * W, C:] = jnp.zeros((W, CQ - C), jnp.float32)
            tg_ref[h * W:(h + 1) * W, :C] = gx_ref[:, h, :].T
            tg_ref[h * W:(h + 1) * W, C:] = gy_ref[:, h, :].T

    grid = (H // HB,)
    return pl.pallas_call(
        body,
        grid=grid,
        in_specs=[
            pl.BlockSpec((C, HB, W), lambda i: (0, i, 0)),
            pl.BlockSpec((C, HB, W), lambda i: (0, i, 0)),
            pl.BlockSpec((C, HB, W), lambda i: (0, i, 0)),
        ],
        out_specs=[
            pl.BlockSpec((HB * W, CQ), lambda i: (i, 0)),
            pl.BlockSpec((HB * W, 2 * C), lambda i: (i, 0)),
        ],
        out_shape=[
            jax.ShapeDtypeStruct((S, CQ), jnp.float32),
            jax.ShapeDtypeStruct((S, 2 * C), jnp.float32),
        ],
    )


def _project16(px, py, pz, ps, H, W, sub):
    (r00, r01, r02, r10, r11, r12, r20, r21, r22,
     t0, t1, t2, k00, k01, k02, k10, k11, k12, k20, k21, k22) = ps
    X = px * r00 + py * r01 + pz * r02 + t0
    Y = px * r10 + py * r11 + pz * r12 + t1
    Z = px * r20 + py * r21 + pz * r22 + t2
    h0 = X * k00 + Y * k10 + Z * k20
    h1 = X * k01 + Y * k11 + Z * k21
    h2 = X * k02 + Y * k12 + Z * k22
    u = h0 / h2
    v = h1 / h2
    iu = u.astype(jnp.int32) - sub
    iv = v.astype(jnp.int32) - sub
    ii = jnp.minimum(jnp.maximum(iu, 0), H - 1)
    jj = jnp.minimum(jnp.maximum(iv, 0), W - 1)
    return ii * W + jj


def _load_pose(pose_v):
    return [pose_v[kk, pl.ds(0, 16)] for kk in range(21)]


def _pose_mat(R, t, Kf):
    vals = jnp.concatenate([R.reshape(-1), t, Kf,
                            jnp.zeros((3,), jnp.float32)])
    return jnp.broadcast_to(vals[:, None], (24, 128))


@functools.lru_cache(maxsize=None)
def _make_sc_gather_qg(N, C, S, CQ, H, W):
    BPW = N // NW
    NG = BPW // 16
    mesh = plsc.VectorSubcoreMesh(core_axis_name="c", subcore_axis_name="s")
    info = plsc.get_sparse_core_info()
    NC = info.num_cores

    @functools.partial(
        pl.kernel,
        mesh=mesh,
        out_type=[
            jax.ShapeDtypeStruct((N, CQ), jnp.float32),
            jax.ShapeDtypeStruct((N, 2 * C), jnp.float32),
        ],
        scratch_types=[
            pltpu.VMEM((24, 128), jnp.float32),
            pltpu.VMEM((BPW,), jnp.float32),
            pltpu.VMEM((BPW,), jnp.float32),
            pltpu.VMEM((BPW,), jnp.float32),
            pltpu.VMEM((BPW,), jnp.int32),
            pltpu.VMEM((BPW, CQ), jnp.float32),
            pltpu.VMEM((BPW, 2 * C), jnp.float32),
            pltpu.SemaphoreType.DMA,
            pltpu.SemaphoreType.DMA,
        ],
    )
    def k(tq_hbm, tg_hbm, x_hbm, y_hbm, z_hbm, pose_hbm, outq_hbm, outg_hbm,
          pose_v, x_v, y_v, z_v, idx_v, q_v, g_v, sem1, sem2):
        wid = lax.axis_index("s") * NC + lax.axis_index("c")
        base = wid * BPW
        pltpu.sync_copy(pose_hbm, pose_v)
        pltpu.sync_copy(x_hbm.at[pl.ds(base, BPW)], x_v)
        pltpu.sync_copy(y_hbm.at[pl.ds(base, BPW)], y_v)
        pltpu.sync_copy(z_hbm.at[pl.ds(base, BPW)], z_v)
        ps = _load_pose(pose_v)
        for g in range(NG):
            sl = pl.ds(g * 16, 16)
            idx_v[sl] = _project16(x_v[sl], y_v[sl], z_v[sl], ps, H, W, 1)
        cq = pltpu.async_copy(tq_hbm.at[idx_v], q_v, sem1)
        cg = pltpu.async_copy(tg_hbm.at[idx_v], g_v, sem2)
        cq.wait()
        cg.wait()
        pltpu.sync_copy(q_v, outq_hbm.at[pl.ds(base, BPW)])
        pltpu.sync_copy(g_v, outg_hbm.at[pl.ds(base, BPW)])

    return k


@functools.lru_cache(maxsize=None)
def _make_sc_gather_q(N, C, S, CQ, H, W):
    BPW = N // NW
    NG = BPW // 16
    mesh = plsc.VectorSubcoreMesh(core_axis_name="c", subcore_axis_name="s")
    info = plsc.get_sparse_core_info()
    NC = info.num_cores

    @functools.partial(
        pl.kernel,
        mesh=mesh,
        out_type=jax.ShapeDtypeStruct((N, CQ), jnp.float32),
        scratch_types=[
            pltpu.VMEM((24, 128), jnp.float32),
            pltpu.VMEM((BPW,), jnp.float32),
            pltpu.VMEM((BPW,), jnp.float32),
            pltpu.VMEM((BPW,), jnp.float32),
            pltpu.VMEM((BPW,), jnp.int32),
            pltpu.VMEM((BPW, CQ), jnp.float32),
            pltpu.SemaphoreType.DMA,
        ],
    )
    def k(tq_hbm, x_hbm, y_hbm, z_hbm, pose_hbm, outq_hbm,
          pose_v, x_v, y_v, z_v, idx_v, q_v, sem1):
        wid = lax.axis_index("s") * NC + lax.axis_index("c")
        base = wid * BPW
        pltpu.sync_copy(pose_hbm, pose_v)
        pltpu.sync_copy(x_hbm.at[pl.ds(base, BPW)], x_v)
        pltpu.sync_copy(y_hbm.at[pl.ds(base, BPW)], y_v)
        pltpu.sync_copy(z_hbm.at[pl.ds(base, BPW)], z_v)
        ps = _load_pose(pose_v)
        for g in range(NG):
            sl = pl.ds(g * 16, 16)
            idx_v[sl] = _project16(x_v[sl], y_v[sl], z_v[sl], ps, H, W, 0)
        pltpu.async_copy(tq_hbm.at[idx_v], q_v, sem1).wait()
        pltpu.sync_copy(q_v, outq_hbm.at[pl.ds(base, BPW)])

    return k


@functools.lru_cache(maxsize=None)
def _make_reduce1(N, C, CQ, NB):
    B = N // NB

    def body(gq_ref, gg_ref, fr_ref, pts_ref, pose_ref, out_ref):
        b = pl.program_id(0)
        q = gq_ref[:, :C]
        gx = gg_ref[:, :C]
        gy = gg_ref[:, C:]
        f = fr_ref[...]
        err = q - f
        sgx = jnp.sum(err * gx, axis=-1)
        sgy = jnp.sum(err * gy, axis=-1)
        wxx = jnp.sum(gx * gx, axis=-1)
        wxy = jnp.sum(gx * gy, axis=-1)
        wyy = jnp.sum(gy * gy, axis=-1)
        ee = jnp.sum(err * err, axis=-1)
        px = pts_ref[0, :]
        py = pts_ref[1, :]
        pz = pts_ref[2, :]
        r00, r01, r02 = pose_ref[0], pose_ref[1], pose_ref[2]
        r10, r11, r12 = pose_ref[3], pose_ref[4], pose_ref[5]
        r20, r21, r22 = pose_ref[6], pose_ref[7], pose_ref[8]
        t0, t1, t2 = pose_ref[9], pose_ref[10], pose_ref[11]
        x = px * r00 + py * r01 + pz * r02 + t0
        y = px * r10 + py * r11 + pz * r12 + t1
        z = px * r20 + py * r21 + pz * r22 + t2
        iz = 1.0 / z
        izz = iz * iz
        zero = jnp.zeros_like(x)
        one = jnp.ones_like(x)
        a00, a01, a02 = iz, zero, -x * izz
        a03, a04, a05 = -x * y * izz, 1.0 + x * x * izz, -y * iz
        a10, a11, a12 = zero, iz, -y * izz
        a13, a14, a15 = -1.0 - y * y * izz, x * y * izz, x * iz
        A0T = jnp.stack([a00, a01, a02, a03, a04, a05, zero, ee], axis=0)
        A1T = jnp.stack([a10, a11, a12, a13, a14, a15, zero, zero], axis=0)
        UT = jnp.stack([
            wxx * a00 + wxy * a10, wxx * a01 + wxy * a11,
            wxx * a02 + wxy * a12, wxx * a03 + wxy * a13,
            wxx * a04 + wxy * a14, wxx * a05 + wxy * a15,
            sgx, one,
        ], axis=0)
        VT = jnp.stack([
            wxy * a00 + wyy * a10, wxy * a01 + wyy * a11,
            wxy * a02 + wyy * a12, wxy * a03 + wyy * a13,
            wxy * a04 + wyy * a14, wxy * a05 + wyy * a15,
            sgy, zero,
        ], axis=0)
        dn = (((1,), (1,)), ((), ()))
        part = (lax.dot_general(A0T, UT, dn, preferred_element_type=jnp.float32)
                + lax.dot_general(A1T, VT, dn, preferred_element_type=jnp.float32))

        @pl.when(b == 0)
        def _():
            out_ref[...] = jnp.zeros_like(out_ref)

        out_ref[...] += part

    return pl.pallas_call(
        body,
        grid=(NB,),
        in_specs=[
            pl.BlockSpec((B, CQ), lambda b: (b, 0)),
            pl.BlockSpec((B, 2 * C), lambda b: (b, 0)),
            pl.BlockSpec((B, C), lambda b: (b, 0)),
            pl.BlockSpec((3, B), lambda b: (0, b)),
            pl.BlockSpec(memory_space=pltpu.SMEM),
        ],
        out_specs=pl.BlockSpec((8, 8), lambda b: (0, 0)),
        out_shape=jax.ShapeDtypeStruct((8, 8), jnp.float32),
    )


@functools.lru_cache(maxsize=None)
def _make_reduce2(N, C, CQ, NB):
    B = N // NB

    def body(gq_ref, fr_ref, out_ref):
        b = pl.program_id(0)
        err = gq_ref[:, :C] - fr_ref[...]
        part = jnp.sum(err * err)

        @pl.when(b == 0)
        def _():
            out_ref[0, 0] = 0.0

        out_ref[0, 0] += part

    return pl.pallas_call(
        body,
        grid=(NB,),
        in_specs=[
            pl.BlockSpec((B, CQ), lambda b: (b, 0)),
            pl.BlockSpec((B, C), lambda b: (b, 0)),
        ],
        out_specs=pl.BlockSpec(memory_space=pltpu.SMEM),
        out_shape=jax.ShapeDtypeStruct((1, 1), jnp.float32),
    )


def _skew(v):
    z = jnp.zeros_like(v[..., 0])
    M = jnp.stack([z, -v[..., 2], v[..., 1],
                   v[..., 2], z, -v[..., 0],
                   -v[..., 1], v[..., 0], z], axis=-1)
    return M.reshape(v.shape[:-1] + (3, 3))


def _so3exp(w):
    theta = jnp.linalg.norm(w)
    small = theta < 1e-7
    ts = jnp.where(small, 1.0, theta)
    Wm = _skew(w)
    I = jnp.eye(3, dtype=w.dtype)
    R = I + jnp.sin(ts) / ts * Wm + (1.0 - jnp.cos(ts)) / (ts * ts) * (Wm @ Wm)
    return jnp.where(small, I + Wm, R)


def _lm_step(g, H, lambda_):
    D = jnp.diag(jnp.diagonal(H) + 1e-09)
    H = H + D * lambda_
    P = jnp.linalg.inv(H)
    return -(P @ g[..., None])[..., 0]


def kernel(pts3D, feature_ref, feature_map_query, feature_grad_x,
           feature_grad_y, K):
    N, C = feature_ref.shape
    _, H, W = feature_map_query.shape
    S = H * W
    CQ = ((C + 127) // 128) * 128

    prep = _make_prep(C, H, W, 8, CQ)
    Tq, Tg = prep(feature_map_query, feature_grad_x, feature_grad_y)

    gather_qg = _make_sc_gather_qg(N, C, S, CQ, H, W)
    gather_q = _make_sc_gather_q(N, C, S, CQ, H, W)
    reduce1 = _make_reduce1(N, C, CQ, 8)
    reduce2 = _make_reduce2(N, C, CQ, 8)

    xs = pts3D[:, 0]
    ys = pts3D[:, 1]
    zs = pts3D[:, 2]
    ptsT = pts3D.T

    R = jnp.eye(3, dtype=jnp.float32)
    t = jnp.array([1.0, 1.0, 0.0], dtype=jnp.float32)
    lam = jnp.asarray(0.01, dtype=jnp.float32)
    Kf = K.reshape(-1)
    prev_cost = None

    for it in range(3):
        pose = _pose_mat(R, t, Kf)
        pose_s = jnp.concatenate([R.reshape(-1), t,
                                  jnp.zeros((4,), jnp.float32)])
        Gq, Gg = gather_qg(Tq, Tg, xs, ys, zs, pose)
        out8 = reduce1(Gq, Gg, feature_ref, ptsT, pose_s)
        Hess = out8[:6, :6]
        Grad = out8[:6, 6]
        if it == 0:
            prev_cost = 0.5 * out8[7, 7] / N
        delta = _lm_step(Grad, Hess, lam)
        dt, dw = delta[:3], delta[3:6]
        dr = _so3exp(dw)
        R_new = dr @ R
        t_new = dr @ t + dt
        pose_new = _pose_mat(R_new, t_new, Kf)
        Gq2 = gather_q(Tq, xs, ys, zs, pose_new)
        new_cost = reduce2(Gq2, feature_ref)[0, 0] / N
        increased = new_cost > prev_cost
        lam = jnp.clip(lam * jnp.where(increased, 10.0, 0.1), 1e-06, 100.0)
        accept = jnp.logical_not(increased)
        prev_cost = jnp.where(accept, new_cost, prev_cost)
        R = jnp.where(accept, R_new, R)
        t = jnp.where(accept, t_new, t)
    return R, t


# speedup vs baseline: 1.0693x; 1.0693x over previous
import functools

import jax
import jax.numpy as jnp
from jax import lax
from jax.experimental import pallas as pl
from jax.experimental.pallas import tpu as pltpu
from jax.experimental.pallas import tpu_sc as plsc

NW = 32


@functools.lru_cache(maxsize=None)
def _make_prep(C, H, W, HB, CQ):
    S = H * W

    def body(q_ref, gx_ref, gy_ref, tq_ref, tg_ref):
        for h in range(HB):
            tq_ref[h * W:(h + 1) * W, :C] = q_ref[:, h, :].T
            tq_ref[h * W:(h + 1) * W, C:] = jnp.zeros((W, CQ - C), jnp.float32)
            tg_ref[h * W:(h + 1) * W, :C] = gx_ref[:, h, :].T
            tg_ref[h * W:(h + 1) * W, C:] = gy_ref[:, h, :].T

    grid = (H // HB,)
    return pl.pallas_call(
        body,
        grid=grid,
        in_specs=[
            pl.BlockSpec((C, HB, W), lambda i: (0, i, 0)),
            pl.BlockSpec((C, HB, W), lambda i: (0, i, 0)),
            pl.BlockSpec((C, HB, W), lambda i: (0, i, 0)),
        ],
        out_specs=[
            pl.BlockSpec((HB * W, CQ), lambda i: (i, 0)),
            pl.BlockSpec((HB * W, 2 * C), lambda i: (i, 0)),
        ],
        out_shape=[
            jax.ShapeDtypeStruct((S, CQ), jnp.float32),
            jax.ShapeDtypeStruct((S, 2 * C), jnp.float32),
        ],
    )


def _project16(px, py, pz, ps, H, W, sub):
    (r00, r01, r02, r10, r11, r12, r20, r21, r22,
     t0, t1, t2, k00, k01, k02, k10, k11, k12, k20, k21, k22) = ps
    X = px * r00 + py * r01 + pz * r02 + t0
    Y = px * r10 + py * r11 + pz * r12 + t1
    Z = px * r20 + py * r21 + pz * r22 + t2
    h0 = X * k00 + Y * k10 + Z * k20
    h1 = X * k01 + Y * k11 + Z * k21
    h2 = X * k02 + Y * k12 + Z * k22
    u = h0 / h2
    v = h1 / h2
    iu = u.astype(jnp.int32) - sub
    iv = v.astype(jnp.int32) - sub
    ii = jnp.minimum(jnp.maximum(iu, 0), H - 1)
    jj = jnp.minimum(jnp.maximum(iv, 0), W - 1)
    return ii * W + jj


def _load_pose(pose_v):
    return [pose_v[kk, pl.ds(0, 16)] for kk in range(21)]


def _pose_mat(R, t, Kf):
    vals = jnp.concatenate([R.reshape(-1), t, Kf,
                            jnp.zeros((3,), jnp.float32)])
    return jnp.broadcast_to(vals[:, None], (24, 128))


@functools.lru_cache(maxsize=None)
def _make_sc_gather_qg(N, C, S, CQ, H, W):
    BPW = N // NW
    NG = BPW // 16
    mesh = plsc.VectorSubcoreMesh(core_axis_name="c", subcore_axis_name="s")
    info = plsc.get_sparse_core_info()
    NC = info.num_cores

    @functools.partial(
        pl.kernel,
        mesh=mesh,
        out_type=[
            jax.ShapeDtypeStruct((N, CQ), jnp.float32),
            jax.ShapeDtypeStruct((N, 2 * C), jnp.float32),
        ],
        scratch_types=[
            pltpu.VMEM((24, 128), jnp.float32),
            pltpu.VMEM((BPW,), jnp.float32),
            pltpu.VMEM((BPW,), jnp.float32),
            pltpu.VMEM((BPW,), jnp.float32),
            pltpu.VMEM((BPW,), jnp.int32),
            pltpu.VMEM((BPW, CQ), jnp.float32),
            pltpu.VMEM((BPW, 2 * C), jnp.float32),
            pltpu.SemaphoreType.DMA,
            pltpu.SemaphoreType.DMA,
        ],
    )
    def k(tq_hbm, tg_hbm, x_hbm, y_hbm, z_hbm, pose_hbm, outq_hbm, outg_hbm,
          pose_v, x_v, y_v, z_v, idx_v, q_v, g_v, sem1, sem2):
        wid = lax.axis_index("s") * NC + lax.axis_index("c")
        base = wid * BPW
        pltpu.sync_copy(pose_hbm, pose_v)
        pltpu.sync_copy(x_hbm.at[pl.ds(base, BPW)], x_v)
        pltpu.sync_copy(y_hbm.at[pl.ds(base, BPW)], y_v)
        pltpu.sync_copy(z_hbm.at[pl.ds(base, BPW)], z_v)
        ps = _load_pose(pose_v)
        for g in range(NG):
            sl = pl.ds(g * 16, 16)
            idx_v[sl] = _project16(x_v[sl], y_v[sl], z_v[sl], ps, H, W, 1)
        cq = pltpu.async_copy(tq_hbm.at[idx_v], q_v, sem1)
        cg = pltpu.async_copy(tg_hbm.at[idx_v], g_v, sem2)
        cq.wait()
        cg.wait()
        pltpu.sync_copy(q_v, outq_hbm.at[pl.ds(base, BPW)])
        pltpu.sync_copy(g_v, outg_hbm.at[pl.ds(base, BPW)])

    return k


@functools.lru_cache(maxsize=None)
def _make_sc_gather_q(N, C, S, CQ, H, W):
    BPW = N // NW
    NG = BPW // 16
    mesh = plsc.VectorSubcoreMesh(core_axis_name="c", subcore_axis_name="s")
    info = plsc.get_sparse_core_info()
    NC = info.num_cores

    @functools.partial(
        pl.kernel,
        mesh=mesh,
        out_type=jax.ShapeDtypeStruct((N, CQ), jnp.float32),
        scratch_types=[
            pltpu.VMEM((24, 128), jnp.float32),
            pltpu.VMEM((BPW,), jnp.float32),
            pltpu.VMEM((BPW,), jnp.float32),
            pltpu.VMEM((BPW,), jnp.float32),
            pltpu.VMEM((BPW,), jnp.int32),
            pltpu.VMEM((BPW, CQ), jnp.float32),
            pltpu.SemaphoreType.DMA,
        ],
    )
    def k(tq_hbm, x_hbm, y_hbm, z_hbm, pose_hbm, outq_hbm,
          pose_v, x_v, y_v, z_v, idx_v, q_v, sem1):
        wid = lax.axis_index("s") * NC + lax.axis_index("c")
        base = wid * BPW
        pltpu.sync_copy(pose_hbm, pose_v)
        pltpu.sync_copy(x_hbm.at[pl.ds(base, BPW)], x_v)
        pltpu.sync_copy(y_hbm.at[pl.ds(base, BPW)], y_v)
        pltpu.sync_copy(z_hbm.at[pl.ds(base, BPW)], z_v)
        ps = _load_pose(pose_v)
        for g in range(NG):
            sl = pl.ds(g * 16, 16)
            idx_v[sl] = _project16(x_v[sl], y_v[sl], z_v[sl], ps, H, W, 0)
        pltpu.async_copy(tq_hbm.at[idx_v], q_v, sem1).wait()
        pltpu.sync_copy(q_v, outq_hbm.at[pl.ds(base, BPW)])

    return k


@functools.lru_cache(maxsize=None)
def _make_reduce1(N, C, CQ, NB):
    B = N // NB

    def body(gq_ref, gg_ref, fr_ref, pts_ref, pose_ref, out_ref):
        b = pl.program_id(0)
        q = gq_ref[:, :C]
        gx = gg_ref[:, :C]
        gy = gg_ref[:, C:]
        f = fr_ref[...]
        err = q - f
        P = jnp.concatenate(
            [err * gx, err * gy, gx * gx, gx * gy, gy * gy, err * err],
            axis=1)
        kd = lax.broadcasted_iota(jnp.int32, (6 * C, 8), 0) // C
        dd = lax.broadcasted_iota(jnp.int32, (6 * C, 8), 1)
        M = (kd == dd).astype(jnp.float32)
        dn0 = (((1,), (0,)), ((), ()))
        D6 = lax.dot_general(P, M, dn0,
                             preferred_element_type=jnp.float32)
        D6T = D6.T
        sgx = D6T[0, :]
        sgy = D6T[1, :]
        wxx = D6T[2, :]
        wxy = D6T[3, :]
        wyy = D6T[4, :]
        ee = D6T[5, :]
        px = pts_ref[0, :]
        py = pts_ref[1, :]
        pz = pts_ref[2, :]
        r00, r01, r02 = pose_ref[0], pose_ref[1], pose_ref[2]
        r10, r11, r12 = pose_ref[3], pose_ref[4], pose_ref[5]
        r20, r21, r22 = pose_ref[6], pose_ref[7], pose_ref[8]
        t0, t1, t2 = pose_ref[9], pose_ref[10], pose_ref[11]
        x = px * r00 + py * r01 + pz * r02 + t0
        y = px * r10 + py * r11 + pz * r12 + t1
        z = px * r20 + py * r21 + pz * r22 + t2
        iz = 1.0 / z
        izz = iz * iz
        zero = jnp.zeros_like(x)
        one = jnp.ones_like(x)
        a00, a01, a02 = iz, zero, -x * izz
        a03, a04, a05 = -x * y * izz, 1.0 + x * x * izz, -y * iz
        a10, a11, a12 = zero, iz, -y * izz
        a13, a14, a15 = -1.0 - y * y * izz, x * y * izz, x * iz
        A0T = jnp.stack([a00, a01, a02, a03, a04, a05, zero, ee], axis=0)
        A1T = jnp.stack([a10, a11, a12, a13, a14, a15, zero, zero], axis=0)
        UT = jnp.stack([
            wxx * a00 + wxy * a10, wxx * a01 + wxy * a11,
            wxx * a02 + wxy * a12, wxx * a03 + wxy * a13,
            wxx * a04 + wxy * a14, wxx * a05 + wxy * a15,
            sgx, one,
        ], axis=0)
        VT = jnp.stack([
            wxy * a00 + wyy * a10, wxy * a01 + wyy * a11,
            wxy * a02 + wyy * a12, wxy * a03 + wyy * a13,
            wxy * a04 + wyy * a14, wxy * a05 + wyy * a15,
            sgy, zero,
        ], axis=0)
        dn = (((1,), (1,)), ((), ()))
        part = (lax.dot_general(A0T, UT, dn, preferred_element_type=jnp.float32)
                + lax.dot_general(A1T, VT, dn, preferred_element_type=jnp.float32))

        @pl.when(b == 0)
        def _():
            out_ref[...] = jnp.zeros_like(out_ref)

        out_ref[...] += part

    return pl.pallas_call(
        body,
        grid=(NB,),
        in_specs=[
            pl.BlockSpec((B, CQ), lambda b: (b, 0)),
            pl.BlockSpec((B, 2 * C), lambda b: (b, 0)),
            pl.BlockSpec((B, C), lambda b: (b, 0)),
            pl.BlockSpec((3, B), lambda b: (0, b)),
            pl.BlockSpec(memory_space=pltpu.SMEM),
        ],
        out_specs=pl.BlockSpec((8, 8), lambda b: (0, 0)),
        out_shape=jax.ShapeDtypeStruct((8, 8), jnp.float32),
    )


@functools.lru_cache(maxsize=None)
def _make_reduce2(N, C, CQ, NB):
    B = N // NB

    def body(gq_ref, fr_ref, out_ref):
        b = pl.program_id(0)
        err = gq_ref[:, :C] - fr_ref[...]
        P = err * err
        ones = jnp.ones((C, 8), jnp.float32)
        dn0 = (((1,), (0,)), ((), ()))
        v = lax.dot_general(P, ones, dn0,
                            preferred_element_type=jnp.float32)
        part = jnp.sum(v[:, 0])

        @pl.when(b == 0)
        def _():
            out_ref[0, 0] = 0.0

        out_ref[0, 0] += part

    return pl.pallas_call(
        body,
        grid=(NB,),
        in_specs=[
            pl.BlockSpec((B, CQ), lambda b: (b, 0)),
            pl.BlockSpec((B, C), lambda b: (b, 0)),
        ],
        out_specs=pl.BlockSpec(memory_space=pltpu.SMEM),
        out_shape=jax.ShapeDtypeStruct((1, 1), jnp.float32),
    )


def _skew(v):
    z = jnp.zeros_like(v[..., 0])
    M = jnp.stack([z, -v[..., 2], v[..., 1],
                   v[..., 2], z, -v[..., 0],
                   -v[..., 1], v[..., 0], z], axis=-1)
    return M.reshape(v.shape[:-1] + (3, 3))


def _so3exp(w):
    theta = jnp.linalg.norm(w)
    small = theta < 1e-7
    ts = jnp.where(small, 1.0, theta)
    Wm = _skew(w)
    I = jnp.eye(3, dtype=w.dtype)
    R = I + jnp.sin(ts) / ts * Wm + (1.0 - jnp.cos(ts)) / (ts * ts) * (Wm @ Wm)
    return jnp.where(small, I + Wm, R)


def _lm_step(g, H, lambda_):
    D = jnp.diag(jnp.diagonal(H) + 1e-09)
    H = H + D * lambda_
    P = jnp.linalg.inv(H)
    return -(P @ g[..., None])[..., 0]


def kernel(pts3D, feature_ref, feature_map_query, feature_grad_x,
           feature_grad_y, K):
    N, C = feature_ref.shape
    _, H, W = feature_map_query.shape
    S = H * W
    CQ = ((C + 127) // 128) * 128

    prep = _make_prep(C, H, W, 8, CQ)
    Tq, Tg = prep(feature_map_query, feature_grad_x, feature_grad_y)

    gather_qg = _make_sc_gather_qg(N, C, S, CQ, H, W)
    gather_q = _make_sc_gather_q(N, C, S, CQ, H, W)
    reduce1 = _make_reduce1(N, C, CQ, 8)
    reduce2 = _make_reduce2(N, C, CQ, 8)

    xs = pts3D[:, 0]
    ys = pts3D[:, 1]
    zs = pts3D[:, 2]
    ptsT = pts3D.T

    R = jnp.eye(3, dtype=jnp.float32)
    t = jnp.array([1.0, 1.0, 0.0], dtype=jnp.float32)
    lam = jnp.asarray(0.01, dtype=jnp.float32)
    Kf = K.reshape(-1)
    prev_cost = None

    for it in range(3):
        pose = _pose_mat(R, t, Kf)
        pose_s = jnp.concatenate([R.reshape(-1), t,
                                  jnp.zeros((4,), jnp.float32)])
        Gq, Gg = gather_qg(Tq, Tg, xs, ys, zs, pose)
        out8 = reduce1(Gq, Gg, feature_ref, ptsT, pose_s)
        Hess = out8[:6, :6]
        Grad = out8[:6, 6]
        if it == 0:
            prev_cost = 0.5 * out8[7, 7] / N
        delta = _lm_step(Grad, Hess, lam)
        dt, dw = delta[:3], delta[3:6]
        dr = _so3exp(dw)
        R_new = dr @ R
        t_new = dr @ t + dt
        pose_new = _pose_mat(R_new, t_new, Kf)
        Gq2 = gather_q(Tq, xs, ys, zs, pose_new)
        new_cost = reduce2(Gq2, feature_ref)[0, 0] / N
        increased = new_cost > prev_cost
        lam = jnp.clip(lam * jnp.where(increased, 10.0, 0.1), 1e-06, 100.0)
        accept = jnp.logical_not(increased)
        prev_cost = jnp.where(accept, new_cost, prev_cost)
        R = jnp.where(accept, R_new, R)
        t = jnp.where(accept, t_new, t)
    return R, t


# speedup vs baseline: 1.1156x; 1.0433x over previous
import functools

import jax
import jax.numpy as jnp
from jax import lax
from jax.experimental import pallas as pl
from jax.experimental.pallas import tpu as pltpu
from jax.experimental.pallas import tpu_sc as plsc

NW = 32


@functools.lru_cache(maxsize=None)
def _make_prep(C, H, W, HB, CQ):
    S = H * W

    def body(q_ref, gx_ref, gy_ref, tq_ref, tg_ref):
        for h in range(HB):
            tq_ref[h * W:(h + 1) * W, :C] = q_ref[:, h, :].T
            tq_ref[h * W:(h + 1) * W, C:] = jnp.zeros((W, CQ - C), jnp.float32)
            tg_ref[h * W:(h + 1) * W, :C] = gx_ref[:, h, :].T
            tg_ref[h * W:(h + 1) * W, C:] = gy_ref[:, h, :].T

    grid = (H // HB,)
    return pl.pallas_call(
        body,
        grid=grid,
        in_specs=[
            pl.BlockSpec((C, HB, W), lambda i: (0, i, 0)),
            pl.BlockSpec((C, HB, W), lambda i: (0, i, 0)),
            pl.BlockSpec((C, HB, W), lambda i: (0, i, 0)),
        ],
        out_specs=[
            pl.BlockSpec((HB * W, CQ), lambda i: (i, 0)),
            pl.BlockSpec((HB * W, 2 * C), lambda i: (i, 0)),
        ],
        out_shape=[
            jax.ShapeDtypeStruct((S, CQ), jnp.float32),
            jax.ShapeDtypeStruct((S, 2 * C), jnp.float32),
        ],
    )


def _project16(px, py, pz, ps, H, W, sub):
    (r00, r01, r02, r10, r11, r12, r20, r21, r22,
     t0, t1, t2, k00, k01, k02, k10, k11, k12, k20, k21, k22) = ps
    X = px * r00 + py * r01 + pz * r02 + t0
    Y = px * r10 + py * r11 + pz * r12 + t1
    Z = px * r20 + py * r21 + pz * r22 + t2
    h0 = X * k00 + Y * k10 + Z * k20
    h1 = X * k01 + Y * k11 + Z * k21
    h2 = X * k02 + Y * k12 + Z * k22
    u = h0 / h2
    v = h1 / h2
    iu = u.astype(jnp.int32) - sub
    iv = v.astype(jnp.int32) - sub
    ii = jnp.minimum(jnp.maximum(iu, 0), H - 1)
    jj = jnp.minimum(jnp.maximum(iv, 0), W - 1)
    return ii * W + jj


def _load_pose(pose_v):
    return [pose_v[kk, pl.ds(0, 16)] for kk in range(21)]


def _pose_mat(R, t, Kf):
    vals = jnp.concatenate([R.reshape(-1), t, Kf,
                            jnp.zeros((3,), jnp.float32)])
    return jnp.broadcast_to(vals[:, None], (24, 128))


@functools.lru_cache(maxsize=None)
def _make_sc_dots1(N, C, S, CQ, H, W):
    BPW = N // NW
    HP = BPW // 2
    NGH = HP // 16
    mesh = plsc.VectorSubcoreMesh(core_axis_name="c", subcore_axis_name="s")
    info = plsc.get_sparse_core_info()
    NC = info.num_cores

    @functools.partial(
        pl.kernel,
        mesh=mesh,
        out_type=jax.ShapeDtypeStruct((N, 128), jnp.float32),
        scratch_types=[
            pltpu.VMEM((24, 128), jnp.float32),
            pltpu.VMEM((BPW,), jnp.float32),
            pltpu.VMEM((BPW,), jnp.float32),
            pltpu.VMEM((BPW,), jnp.float32),
            pltpu.VMEM((BPW,), jnp.int32),
            pltpu.VMEM((HP, CQ), jnp.float32),
            pltpu.VMEM((HP, 2 * C), jnp.float32),
            pltpu.VMEM((HP, C), jnp.float32),
            pltpu.VMEM((HP, 128), jnp.float32),
            pltpu.SemaphoreType.DMA,
            pltpu.SemaphoreType.DMA,
        ],
    )
    def k(tq_hbm, tg_hbm, fref_hbm, x_hbm, y_hbm, z_hbm, pose_hbm, out_hbm,
          pose_v, x_v, y_v, z_v, idx_v, q_v, g_v, f_v, out_v, sem1, sem2):
        wid = lax.axis_index("s") * NC + lax.axis_index("c")
        base = wid * BPW
        pltpu.sync_copy(pose_hbm, pose_v)
        pltpu.sync_copy(x_hbm.at[pl.ds(base, BPW)], x_v)
        pltpu.sync_copy(y_hbm.at[pl.ds(base, BPW)], y_v)
        pltpu.sync_copy(z_hbm.at[pl.ds(base, BPW)], z_v)
        ps = _load_pose(pose_v)
        for g in range(2 * NGH):
            sl = pl.ds(g * 16, 16)
            idx_v[sl] = _project16(x_v[sl], y_v[sl], z_v[sl], ps, H, W, 1)
        zero16 = jnp.zeros((16,), jnp.float32)
        for half in range(2):
            hb = half * HP
            cq = pltpu.async_copy(
                tq_hbm.at[idx_v.at[pl.ds(hb, HP)]], q_v, sem1)
            cg = pltpu.async_copy(
                tg_hbm.at[idx_v.at[pl.ds(hb, HP)]], g_v, sem2)
            pltpu.sync_copy(fref_hbm.at[pl.ds(base + hb, HP)], f_v)
            cq.wait()
            cg.wait()

            def pbody(p, carry):
                asgx = zero16
                asgy = zero16
                awxx = zero16
                awxy = zero16
                awyy = zero16
                aee = zero16
                for ch in range(C // 16):
                    sl = pl.ds(ch * 16, 16)
                    slg = pl.ds(C + ch * 16, 16)
                    qv = q_v[p, sl]
                    gxv = g_v[p, sl]
                    gyv = g_v[p, slg]
                    fv = f_v[p, sl]
                    errv = qv - fv
                    asgx = asgx + errv * gxv
                    asgy = asgy + errv * gyv
                    awxx = awxx + gxv * gxv
                    awxy = awxy + gxv * gyv
                    awyy = awyy + gyv * gyv
                    aee = aee + errv * errv
                out_v[p, pl.ds(0, 16)] = asgx
                out_v[p, pl.ds(16, 16)] = asgy
                out_v[p, pl.ds(32, 16)] = awxx
                out_v[p, pl.ds(48, 16)] = awxy
                out_v[p, pl.ds(64, 16)] = awyy
                out_v[p, pl.ds(80, 16)] = aee
                out_v[p, pl.ds(96, 16)] = zero16
                out_v[p, pl.ds(112, 16)] = zero16
                return carry

            lax.fori_loop(0, HP, pbody, 0)
            pltpu.sync_copy(out_v, out_hbm.at[pl.ds(base + hb, HP)])

    return k


@functools.lru_cache(maxsize=None)
def _make_sc_dots2(N, C, S, CQ, H, W):
    BPW = N // NW
    NG = BPW // 16
    mesh = plsc.VectorSubcoreMesh(core_axis_name="c", subcore_axis_name="s")
    info = plsc.get_sparse_core_info()
    NC = info.num_cores

    @functools.partial(
        pl.kernel,
        mesh=mesh,
        out_type=jax.ShapeDtypeStruct((NW, 16), jnp.float32),
        scratch_types=[
            pltpu.VMEM((24, 128), jnp.float32),
            pltpu.VMEM((BPW,), jnp.float32),
            pltpu.VMEM((BPW,), jnp.float32),
            pltpu.VMEM((BPW,), jnp.float32),
            pltpu.VMEM((BPW,), jnp.int32),
            pltpu.VMEM((BPW, CQ), jnp.float32),
            pltpu.VMEM((BPW, C), jnp.float32),
            pltpu.VMEM((1, 16), jnp.float32),
            pltpu.SemaphoreType.DMA,
        ],
    )
    def k(tq_hbm, fref_hbm, x_hbm, y_hbm, z_hbm, pose_hbm, out_hbm,
          pose_v, x_v, y_v, z_v, idx_v, q_v, f_v, ee_v, sem1):
        wid = lax.axis_index("s") * NC + lax.axis_index("c")
        base = wid * BPW
        pltpu.sync_copy(pose_hbm, pose_v)
        pltpu.sync_copy(x_hbm.at[pl.ds(base, BPW)], x_v)
        pltpu.sync_copy(y_hbm.at[pl.ds(base, BPW)], y_v)
        pltpu.sync_copy(z_hbm.at[pl.ds(base, BPW)], z_v)
        ps = _load_pose(pose_v)
        for g in range(NG):
            sl = pl.ds(g * 16, 16)
            idx_v[sl] = _project16(x_v[sl], y_v[sl], z_v[sl], ps, H, W, 0)
        cq = pltpu.async_copy(tq_hbm.at[idx_v], q_v, sem1)
        pltpu.sync_copy(fref_hbm.at[pl.ds(base, BPW)], f_v)
        cq.wait()
        zero16 = jnp.zeros((16,), jnp.float32)

        def pbody(p, tot):
            aee = zero16
            for ch in range(C // 16):
                sl = pl.ds(ch * 16, 16)
                errv = q_v[p, sl] - f_v[p, sl]
                aee = aee + errv * errv
            return tot + aee

        total = lax.fori_loop(0, BPW, pbody, zero16)
        ee_v[0, :] = total
        pltpu.sync_copy(ee_v, out_hbm.at[pl.ds(wid, 1), :])

    return k


@functools.lru_cache(maxsize=None)
def _make_reduce1(N, C):
    def body(dots_ref, pts_ref, pose_ref, out_ref):
        kd = lax.broadcasted_iota(jnp.int32, (128, 8), 0) // 16
        dd = lax.broadcasted_iota(jnp.int32, (128, 8), 1)
        Msel = (kd == dd).astype(jnp.float32)
        dn0 = (((1,), (0,)), ((), ()))
        D6 = lax.dot_general(dots_ref[...], Msel, dn0,
                             preferred_element_type=jnp.float32)
        D6T = D6.T
        sgx = D6T[0, :]
        sgy = D6T[1, :]
        wxx = D6T[2, :]
        wxy = D6T[3, :]
        wyy = D6T[4, :]
        ee = D6T[5, :]
        px = pts_ref[0, :]
        py = pts_ref[1, :]
        pz = pts_ref[2, :]
        r00, r01, r02 = pose_ref[0], pose_ref[1], pose_ref[2]
        r10, r11, r12 = pose_ref[3], pose_ref[4], pose_ref[5]
        r20, r21, r22 = pose_ref[6], pose_ref[7], pose_ref[8]
        t0, t1, t2 = pose_ref[9], pose_ref[10], pose_ref[11]
        x = px * r00 + py * r01 + pz * r02 + t0
        y = px * r10 + py * r11 + pz * r12 + t1
        z = px * r20 + py * r21 + pz * r22 + t2
        iz = 1.0 / z
        izz = iz * iz
        zero = jnp.zeros_like(x)
        one = jnp.ones_like(x)
        a00, a01, a02 = iz, zero, -x * izz
        a03, a04, a05 = -x * y * izz, 1.0 + x * x * izz, -y * iz
        a10, a11, a12 = zero, iz, -y * izz
        a13, a14, a15 = -1.0 - y * y * izz, x * y * izz, x * iz
        A0T = jnp.stack([a00, a01, a02, a03, a04, a05, zero, ee], axis=0)
        A1T = jnp.stack([a10, a11, a12, a13, a14, a15, zero, zero], axis=0)
        UT = jnp.stack([
            wxx * a00 + wxy * a10, wxx * a01 + wxy * a11,
            wxx * a02 + wxy * a12, wxx * a03 + wxy * a13,
            wxx * a04 + wxy * a14, wxx * a05 + wxy * a15,
            sgx, one,
        ], axis=0)
        VT = jnp.stack([
            wxy * a00 + wyy * a10, wxy * a01 + wyy * a11,
            wxy * a02 + wyy * a12, wxy * a03 + wyy * a13,
            wxy * a04 + wyy * a14, wxy * a05 + wyy * a15,
            sgy, zero,
        ], axis=0)
        dn = (((1,), (1,)), ((), ()))
        out_ref[...] = (
            lax.dot_general(A0T, UT, dn, preferred_element_type=jnp.float32)
            + lax.dot_general(A1T, VT, dn, preferred_element_type=jnp.float32)
        )

    return pl.pallas_call(
        body,
        in_specs=[
            pl.BlockSpec(memory_space=pltpu.VMEM),
            pl.BlockSpec(memory_space=pltpu.VMEM),
            pl.BlockSpec(memory_space=pltpu.SMEM),
        ],
        out_shape=jax.ShapeDtypeStruct((8, 8), jnp.float32),
    )


def _skew(v):
    z = jnp.zeros_like(v[..., 0])
    M = jnp.stack([z, -v[..., 2], v[..., 1],
                   v[..., 2], z, -v[..., 0],
                   -v[..., 1], v[..., 0], z], axis=-1)
    return M.reshape(v.shape[:-1] + (3, 3))


def _so3exp(w):
    theta = jnp.linalg.norm(w)
    small = theta < 1e-7
    ts = jnp.where(small, 1.0, theta)
    Wm = _skew(w)
    I = jnp.eye(3, dtype=w.dtype)
    R = I + jnp.sin(ts) / ts * Wm + (1.0 - jnp.cos(ts)) / (ts * ts) * (Wm @ Wm)
    return jnp.where(small, I + Wm, R)


def _lm_step(g, H, lambda_):
    D = jnp.diag(jnp.diagonal(H) + 1e-09)
    H = H + D * lambda_
    P = jnp.linalg.inv(H)
    return -(P @ g[..., None])[..., 0]


def kernel(pts3D, feature_ref, feature_map_query, feature_grad_x,
           feature_grad_y, K):
    N, C = feature_ref.shape
    _, H, W = feature_map_query.shape
    S = H * W
    CQ = ((C + 127) // 128) * 128

    prep = _make_prep(C, H, W, 8, CQ)
    Tq, Tg = prep(feature_map_query, feature_grad_x, feature_grad_y)

    sc_dots1 = _make_sc_dots1(N, C, S, CQ, H, W)
    sc_dots2 = _make_sc_dots2(N, C, S, CQ, H, W)
    reduce1 = _make_reduce1(N, C)

    xs = pts3D[:, 0]
    ys = pts3D[:, 1]
    zs = pts3D[:, 2]
    ptsT = pts3D.T

    R = jnp.eye(3, dtype=jnp.float32)
    t = jnp.array([1.0, 1.0, 0.0], dtype=jnp.float32)
    lam = jnp.asarray(0.01, dtype=jnp.float32)
    Kf = K.reshape(-1)
    prev_cost = None

    for it in range(3):
        pose = _pose_mat(R, t, Kf)
        pose_s = jnp.concatenate([R.reshape(-1), t,
                                  jnp.zeros((4,), jnp.float32)])
        dots = sc_dots1(Tq, Tg, feature_ref, xs, ys, zs, pose)
        out8 = reduce1(dots, ptsT, pose_s)
        Hess = out8[:6, :6]
        Grad = out8[:6, 6]
        if it == 0:
            prev_cost = 0.5 * out8[7, 7] / N
        delta = _lm_step(Grad, Hess, lam)
        dt, dw = delta[:3], delta[3:6]
        dr = _so3exp(dw)
        R_new = dr @ R
        t_new = dr @ t + dt
        pose_new = _pose_mat(R_new, t_new, Kf)
        ee2 = sc_dots2(Tq, feature_ref, xs, ys, zs, pose_new)
        new_cost = jnp.sum(ee2) / N
        increased = new_cost > prev_cost
        lam = jnp.clip(lam * jnp.where(increased, 10.0, 0.1), 1e-06, 100.0)
        accept = jnp.logical_not(increased)
        prev_cost = jnp.where(accept, new_cost, prev_cost)
        R = jnp.where(accept, R_new, R)
        t = jnp.where(accept, t_new, t)
    return R, t
